# baseline (speedup 1.0000x reference)
import functools

import jax
import jax.numpy as jnp
from jax import lax
from jax.experimental import pallas as pl
from jax.experimental.pallas import tpu as pltpu
from jax.experimental.pallas import tpu_sc as plsc

L = 16
NUM_WORKERS = 32
B = 128
NQ = 65536
NBLK = NQ // (L * L)
NSB = NBLK // L
TOP_K = 5
ROWS_PER_W = B // NUM_WORKERS
NEG = float("-inf")
BIG = 2.0**30
QPD = 256
CH = NQ // NSB


def _permute(x, idx):
    return lax.gather(
        x, idx[:, None],
        dimension_numbers=lax.GatherDimensionNumbers(
            offset_dims=(), collapsed_slice_dims=(0,), start_index_map=(0,)),
        slice_sizes=(1,),
        mode=lax.GatherScatterMode.PROMISE_IN_BOUNDS,
        unique_indices=True)


def _allmax(x, lane):
    for s in (1, 2, 4, 8):
        x = jnp.maximum(x, _permute(x, lane ^ s))
    return x


def _allmin(x, lane):
    for s in (1, 2, 4, 8):
        x = jnp.minimum(x, _permute(x, lane ^ s))
    return x


def _splat_at(vec, lane, pos):
    return _allmax(
        jnp.where(lane == pos, vec, 0).astype(jnp.float32), lane
    ).astype(jnp.int32)


def _tec_body(cam_hbm, fs_hbm, out_hbm, row_v, lanemax_v, lm2_v, fs_v,
              outst_v, sem):
    wid = lax.axis_index("s") * 2 + lax.axis_index("c")
    lane = lax.iota(jnp.int32, L)

    pltpu.sync_copy(fs_hbm, fs_v.at[pl.ds(0, 4)])
    fsv = fs_v[...]
    hh = _splat_at(fsv, lane, 2)
    ww = _splat_at(fsv, lane, 3)

    def row_body(rr, _):
        row = wid * ROWS_PER_W + rr

        for c in range(NSB):
            pltpu.async_copy(
                cam_hbm.at[row, 1, pl.ds(c * CH, CH)],
                row_v.at[pl.ds(c * CH, CH)], sem)

        def sb_body(s, _):
            pltpu.make_async_copy(
                cam_hbm.at[0, 1, pl.ds(0, CH)],
                row_v.at[pl.ds(0, CH)], sem).wait()

            def blk(b, l2):
                base = b * (L * L)
                vs = [row_v[pl.ds(base + k * L, L)] for k in range(L)]
                while len(vs) > 1:
                    vs = [jnp.maximum(vs[p], vs[p + 1])
                          for p in range(0, len(vs), 2)]
                lanemax_v[pl.ds(b * L, L)] = vs[0]
                return jnp.maximum(l2, vs[0])

            l2 = lax.fori_loop(s * L, (s + 1) * L, blk,
                               jnp.full((L,), NEG, jnp.float32))
            lm2_v[pl.ds(s * L, L)] = l2
            return 0

        lax.fori_loop(0, NSB, sb_body, 0)

        def extract(i, outz):
            def lm2_scan(s, carry):
                cv, cs = carry
                x = lm2_v[pl.ds(s * L, L)]
                m = x > cv
                return jnp.where(m, x, cv), jnp.where(m, s, cs)

            cv, cs = lax.fori_loop(
                0, NSB, lm2_scan,
                (jnp.full((L,), NEG, jnp.float32), jnp.zeros((L,), jnp.int32)))
            gmax = _allmax(cv, lane)
            swin = _allmin(
                jnp.where(cv == gmax, cs.astype(jnp.float32), BIG),
                lane).astype(jnp.int32)

            def bscan(t, bf):
                g_t = plsc.load_gather(
                    lanemax_v, [swin * (L * L) + t * L + lane])
                return jnp.minimum(
                    bf, jnp.where(g_t == gmax, t.astype(jnp.float32), BIG))

            bf = lax.fori_loop(0, L, bscan, jnp.full((L,), BIG, jnp.float32))
            bwin = swin * L + _allmin(bf, lane).astype(jnp.int32)

            def escan(t, best):
                idx_t = bwin * (L * L) + L * lane + t
                vals_t = plsc.load_gather(row_v, [idx_t])
                return jnp.minimum(
                    best, jnp.where(vals_t == gmax,
                                    idx_t.astype(jnp.float32), BIG))

            best = lax.fori_loop(0, L, escan,
                                 jnp.full((L,), BIG, jnp.float32))
            gwin = _allmin(best, lane).astype(jnp.int32)

            gbase = (gwin & ~jnp.int32(255)) | (gwin & 15)
            idxvec = gbase + L * lane
            vals = plsc.load_gather(row_v, [idxvec])
            vals = jnp.where(idxvec == gwin, NEG, vals)
            plsc.store_scatter(row_v, [idxvec], vals)
            m2 = _allmax(vals, lane)
            jhat = gwin & 15
            lmidx = ((gwin >> 8) << 4) + lane
            cur_l = plsc.load_gather(lanemax_v, [lmidx])
            plsc.store_scatter(lanemax_v, [lmidx],
                               jnp.where(lane == jhat, m2, cur_l))
            col = plsc.load_gather(
                lanemax_v, [((gwin >> 12) << 8) + L * lane + jhat])
            m3 = _allmax(col, lane)
            lmidx2 = ((gwin >> 12) << 4) + lane
            cur2 = plsc.load_gather(lm2_v, [lmidx2])
            plsc.store_scatter(lm2_v, [lmidx2],
                               jnp.where(lane == jhat, m3, cur2))

            x_c = gwin % QPD
            y_c = gwin // QPD
            sx = (x_c * ww) // QPD
            sy = (y_c * hh) // QPD
            outz = jnp.where(lane == 2 * i, sx, outz)
            return jnp.where(lane == 2 * i + 1, sy, outz)

        outz = lax.fori_loop(0, TOP_K, extract, jnp.zeros((L,), jnp.int32))
        outst_v[pl.ds(rr * L, L)] = outz
        return 0

    lax.fori_loop(0, ROWS_PER_W, row_body, 0)
    pltpu.sync_copy(outst_v,
                    out_hbm.at[pl.ds(wid * ROWS_PER_W * L, ROWS_PER_W * L)])


@functools.partial(
    pl.kernel,
    out_type=jax.ShapeDtypeStruct((B * L,), jnp.int32),
    mesh=plsc.VectorSubcoreMesh(core_axis_name="c", subcore_axis_name="s"),
    compiler_params=pltpu.CompilerParams(needs_layout_passes=False),
    scratch_types=[
        pltpu.VMEM((NQ,), jnp.float32),
        pltpu.VMEM((NBLK * L,), jnp.float32),
        pltpu.VMEM((NSB * L,), jnp.float32),
        pltpu.VMEM((L,), jnp.int32),
        pltpu.VMEM((ROWS_PER_W * L,), jnp.int32),
        pltpu.SemaphoreType.DMA,
    ],
)
def _sc_topk(cam_hbm, fs_hbm, out_hbm, row_v, lanemax_v, lm2_v, fs_v,
             outst_v, sem):
    _tec_body(cam_hbm, fs_hbm, out_hbm, row_v, lanemax_v, lm2_v, fs_v,
              outst_v, sem)


def kernel(cam, features_shape):
    out = _sc_topk(cam, features_shape.astype(jnp.int32))
    return out.reshape(B, L)[:, : 2 * TOP_K].reshape(B, TOP_K, 2)


# speedup vs baseline: 1.0628x; 1.0628x over previous
import functools

import jax
import jax.numpy as jnp
from jax import lax
from jax.experimental import pallas as pl
from jax.experimental.pallas import tpu as pltpu
from jax.experimental.pallas import tpu_sc as plsc

L = 16
NUM_WORKERS = 32
B = 128
NQ = 65536
NBLK = NQ // (L * L)
NSB = NBLK // L
TOP_K = 5
ROWS_PER_W = B // NUM_WORKERS
NEG = float("-inf")
BIG = 2.0**30
QPD = 256
CH = NQ // NSB


def _permute(x, idx):
    return lax.gather(
        x, idx[:, None],
        dimension_numbers=lax.GatherDimensionNumbers(
            offset_dims=(), collapsed_slice_dims=(0,), start_index_map=(0,)),
        slice_sizes=(1,),
        mode=lax.GatherScatterMode.PROMISE_IN_BOUNDS,
        unique_indices=True)


def _allmax(x, lane):
    for s in (1, 2, 4, 8):
        x = jnp.maximum(x, _permute(x, lane ^ s))
    return x


def _allmin(x, lane):
    for s in (1, 2, 4, 8):
        x = jnp.minimum(x, _permute(x, lane ^ s))
    return x


def _splat_at(vec, lane, pos):
    return _allmax(
        jnp.where(lane == pos, vec, 0).astype(jnp.float32), lane
    ).astype(jnp.int32)


def _tec_body(cam_hbm, fs_hbm, out_hbm, row_v, lanemax_v, lm2_v, fs_v,
              outst_v, sem):
    wid = lax.axis_index("s") * 2 + lax.axis_index("c")
    lane = lax.iota(jnp.int32, L)

    pltpu.sync_copy(fs_hbm, fs_v.at[pl.ds(0, 4)])
    fsv = fs_v[...]
    hh = _splat_at(fsv, lane, 2)
    ww = _splat_at(fsv, lane, 3)

    def row_body(rr, _):
        row = wid * ROWS_PER_W + rr

        for c in range(NSB):
            pltpu.async_copy(
                cam_hbm.at[row, 1, pl.ds(c * CH, CH)],
                row_v.at[pl.ds(c * CH, CH)], sem)

        def sb_body(s, _):
            pltpu.make_async_copy(
                cam_hbm.at[0, 1, pl.ds(0, CH)],
                row_v.at[pl.ds(0, CH)], sem).wait()

            def blk(b, l2):
                base = b * (L * L)
                vs = [row_v[pl.ds(base + k * L, L)] for k in range(L)]
                while len(vs) > 1:
                    vs = [jnp.maximum(vs[p], vs[p + 1])
                          for p in range(0, len(vs), 2)]
                lanemax_v[pl.ds(b * L, L)] = vs[0]
                return jnp.maximum(l2, vs[0])

            l2 = lax.fori_loop(s * L, (s + 1) * L, blk,
                               jnp.full((L,), NEG, jnp.float32))
            lm2_v[pl.ds(s * L, L)] = l2
            return 0

        lax.fori_loop(0, NSB, sb_body, 0)

        def extract(i, outz):
            cv = jnp.full((L,), NEG, jnp.float32)
            cs = jnp.zeros((L,), jnp.int32)
            for s in range(NSB):
                x = lm2_v[pl.ds(s * L, L)]
                m = x > cv
                cv = jnp.where(m, x, cv)
                cs = jnp.where(m, s, cs)
            gmax = _allmax(cv, lane)
            swin = _allmin(
                jnp.where(cv == gmax, cs.astype(jnp.float32), BIG),
                lane).astype(jnp.int32)

            bf = jnp.full((L,), BIG, jnp.float32)
            for t in range(L):
                g_t = plsc.load_gather(
                    lanemax_v, [swin * (L * L) + t * L + lane])
                bf = jnp.minimum(bf, jnp.where(g_t == gmax, float(t), BIG))
            bwin = swin * L + _allmin(bf, lane).astype(jnp.int32)

            best = jnp.full((L,), BIG, jnp.float32)
            for t in range(L):
                idx_t = bwin * (L * L) + L * lane + t
                vals_t = plsc.load_gather(row_v, [idx_t])
                best = jnp.minimum(
                    best, jnp.where(vals_t == gmax,
                                    idx_t.astype(jnp.float32), BIG))
            gwin = _allmin(best, lane).astype(jnp.int32)

            gbase = (gwin & ~jnp.int32(255)) | (gwin & 15)
            idxvec = gbase + L * lane
            vals = plsc.load_gather(row_v, [idxvec])
            vals = jnp.where(idxvec == gwin, NEG, vals)
            plsc.store_scatter(row_v, [idxvec], vals)
            m2 = _allmax(vals, lane)
            jhat = gwin & 15
            lmidx = ((gwin >> 8) << 4) + lane
            cur_l = plsc.load_gather(lanemax_v, [lmidx])
            plsc.store_scatter(lanemax_v, [lmidx],
                               jnp.where(lane == jhat, m2, cur_l))
            col = plsc.load_gather(
                lanemax_v, [((gwin >> 12) << 8) + L * lane + jhat])
            m3 = _allmax(col, lane)
            lmidx2 = ((gwin >> 12) << 4) + lane
            cur2 = plsc.load_gather(lm2_v, [lmidx2])
            plsc.store_scatter(lm2_v, [lmidx2],
                               jnp.where(lane == jhat, m3, cur2))

            x_c = gwin % QPD
            y_c = gwin // QPD
            sx = (x_c * ww) // QPD
            sy = (y_c * hh) // QPD
            outz = jnp.where(lane == 2 * i, sx, outz)
            return jnp.where(lane == 2 * i + 1, sy, outz)

        outz = lax.fori_loop(0, TOP_K, extract, jnp.zeros((L,), jnp.int32))
        outst_v[pl.ds(rr * L, L)] = outz
        return 0

    lax.fori_loop(0, ROWS_PER_W, row_body, 0)
    pltpu.sync_copy(outst_v,
                    out_hbm.at[pl.ds(wid * ROWS_PER_W * L, ROWS_PER_W * L)])


@functools.partial(
    pl.kernel,
    out_type=jax.ShapeDtypeStruct((B * L,), jnp.int32),
    mesh=plsc.VectorSubcoreMesh(core_axis_name="c", subcore_axis_name="s"),
    compiler_params=pltpu.CompilerParams(needs_layout_passes=False),
    scratch_types=[
        pltpu.VMEM((NQ,), jnp.float32),
        pltpu.VMEM((NBLK * L,), jnp.float32),
        pltpu.VMEM((NSB * L,), jnp.float32),
        pltpu.VMEM((L,), jnp.int32),
        pltpu.VMEM((ROWS_PER_W * L,), jnp.int32),
        pltpu.SemaphoreType.DMA,
    ],
)
def _sc_topk(cam_hbm, fs_hbm, out_hbm, row_v, lanemax_v, lm2_v, fs_v,
             outst_v, sem):
    _tec_body(cam_hbm, fs_hbm, out_hbm, row_v, lanemax_v, lm2_v, fs_v,
              outst_v, sem)


def kernel(cam, features_shape):
    out = _sc_topk(cam, features_shape.astype(jnp.int32))
    return out.reshape(B, L)[:, : 2 * TOP_K].reshape(B, TOP_K, 2)


# speedup vs baseline: 1.1599x; 1.0914x over previous
import functools

import jax
import jax.numpy as jnp
from jax import lax
from jax.experimental import pallas as pl
from jax.experimental.pallas import tpu as pltpu
from jax.experimental.pallas import tpu_sc as plsc

L = 16
NUM_WORKERS = 32
B = 128
NQ = 65536
NBLK = NQ // (L * L)
NSB = NBLK // L
TOP_K = 5
ROWS_PER_W = B // NUM_WORKERS
NEG = float("-inf")
BIG = 2.0**30
QPD = 256
CH = NQ // NSB


def _permute(x, idx):
    return lax.gather(
        x, idx[:, None],
        dimension_numbers=lax.GatherDimensionNumbers(
            offset_dims=(), collapsed_slice_dims=(0,), start_index_map=(0,)),
        slice_sizes=(1,),
        mode=lax.GatherScatterMode.PROMISE_IN_BOUNDS,
        unique_indices=True)


def _allmax(x, lane):
    for s in (1, 2, 4, 8):
        x = jnp.maximum(x, _permute(x, lane ^ s))
    return x


def _allmin(x, lane):
    for s in (1, 2, 4, 8):
        x = jnp.minimum(x, _permute(x, lane ^ s))
    return x


def _splat_at(vec, lane, pos):
    return _allmax(
        jnp.where(lane == pos, vec, 0).astype(jnp.float32), lane
    ).astype(jnp.int32)


def _tec_body(cam_hbm, fs_hbm, out_hbm, row_v, lanemax_v, lm2_v, fs_v,
              outst_v, sem):
    wid = lax.axis_index("s") * 2 + lax.axis_index("c")
    lane = lax.iota(jnp.int32, L)

    pltpu.sync_copy(fs_hbm, fs_v.at[pl.ds(0, 4)])
    fsv = fs_v[...]
    hh = _splat_at(fsv, lane, 2)
    ww = _splat_at(fsv, lane, 3)

    def row_body(rr, _):
        row = wid * ROWS_PER_W + rr

        for c in range(NSB):
            pltpu.async_copy(
                cam_hbm.at[row, 1, pl.ds(c * CH, CH)],
                row_v.at[pl.ds(c * CH, CH)], sem)

        def sb_body(s, _):
            pltpu.make_async_copy(
                cam_hbm.at[0, 1, pl.ds(0, CH)],
                row_v.at[pl.ds(0, CH)], sem).wait()

            @plsc.parallel_loop(s * L, (s + 1) * L, step=1, unroll=2,
                                carry=jnp.full((L,), NEG, jnp.float32))
            def blk(b, l2):
                base = b * (L * L)
                vs = [row_v[pl.ds(base + k * L, L)] for k in range(L)]
                while len(vs) > 1:
                    vs = [jnp.maximum(vs[p], vs[p + 1])
                          for p in range(0, len(vs), 2)]
                lanemax_v[pl.ds(b * L, L)] = vs[0]
                return jnp.maximum(l2, vs[0])

            lm2_v[pl.ds(s * L, L)] = blk
            return 0

        lax.fori_loop(0, NSB, sb_body, 0)

        def extract(i, outz):
            cv = jnp.full((L,), NEG, jnp.float32)
            cs = jnp.zeros((L,), jnp.int32)
            for s in range(NSB):
                x = lm2_v[pl.ds(s * L, L)]
                m = x > cv
                cv = jnp.where(m, x, cv)
                cs = jnp.where(m, s, cs)
            gmax = _allmax(cv, lane)
            swin = _allmin(
                jnp.where(cv == gmax, cs.astype(jnp.float32), BIG),
                lane).astype(jnp.int32)

            bf = jnp.full((L,), BIG, jnp.float32)
            for t in range(L):
                g_t = plsc.load_gather(
                    lanemax_v, [swin * (L * L) + t * L + lane])
                bf = jnp.minimum(bf, jnp.where(g_t == gmax, float(t), BIG))
            bwin = swin * L + _allmin(bf, lane).astype(jnp.int32)

            best = jnp.full((L,), BIG, jnp.float32)
            for t in range(L):
                idx_t = bwin * (L * L) + L * lane + t
                vals_t = plsc.load_gather(row_v, [idx_t])
                best = jnp.minimum(
                    best, jnp.where(vals_t == gmax,
                                    idx_t.astype(jnp.float32), BIG))
            gwin = _allmin(best, lane).astype(jnp.int32)

            gbase = (gwin & ~jnp.int32(255)) | (gwin & 15)
            idxvec = gbase + L * lane
            vals = plsc.load_gather(row_v, [idxvec])
            vals = jnp.where(idxvec == gwin, NEG, vals)
            plsc.store_scatter(row_v, [idxvec], vals)
            m2 = _allmax(vals, lane)
            jhat = gwin & 15
            lmidx = ((gwin >> 8) << 4) + lane
            cur_l = plsc.load_gather(lanemax_v, [lmidx])
            plsc.store_scatter(lanemax_v, [lmidx],
                               jnp.where(lane == jhat, m2, cur_l))
            col = plsc.load_gather(
                lanemax_v, [((gwin >> 12) << 8) + L * lane + jhat])
            m3 = _allmax(col, lane)
            lmidx2 = ((gwin >> 12) << 4) + lane
            cur2 = plsc.load_gather(lm2_v, [lmidx2])
            plsc.store_scatter(lm2_v, [lmidx2],
                               jnp.where(lane == jhat, m3, cur2))

            x_c = gwin % QPD
            y_c = gwin // QPD
            sx = (x_c * ww) // QPD
            sy = (y_c * hh) // QPD
            outz = jnp.where(lane == 2 * i, sx, outz)
            return jnp.where(lane == 2 * i + 1, sy, outz)

        outz = lax.fori_loop(0, TOP_K, extract, jnp.zeros((L,), jnp.int32))
        plsc.store_scatter(outst_v, [rr * (2 * TOP_K) + lane], outz)
        return 0

    lax.fori_loop(0, ROWS_PER_W, row_body, 0)
    opw = ROWS_PER_W * 2 * TOP_K
    pltpu.sync_copy(outst_v.at[pl.ds(0, opw)],
                    out_hbm.at[pl.ds(wid * opw, opw)])


@functools.partial(
    pl.kernel,
    out_type=jax.ShapeDtypeStruct((B * 2 * TOP_K,), jnp.int32),
    mesh=plsc.VectorSubcoreMesh(core_axis_name="c", subcore_axis_name="s"),
    compiler_params=pltpu.CompilerParams(needs_layout_passes=False),
    scratch_types=[
        pltpu.VMEM((NQ,), jnp.float32),
        pltpu.VMEM((NBLK * L,), jnp.float32),
        pltpu.VMEM((NSB * L,), jnp.float32),
        pltpu.VMEM((L,), jnp.int32),
        pltpu.VMEM((ROWS_PER_W * 2 * TOP_K + L,), jnp.int32),
        pltpu.SemaphoreType.DMA,
    ],
)
def _sc_topk(cam_hbm, fs_hbm, out_hbm, row_v, lanemax_v, lm2_v, fs_v,
             outst_v, sem):
    _tec_body(cam_hbm, fs_hbm, out_hbm, row_v, lanemax_v, lm2_v, fs_v,
              outst_v, sem)


def kernel(cam, features_shape):
    out = _sc_topk(cam, features_shape.astype(jnp.int32))
    return out.reshape(B, TOP_K, 2)
